# baseline (speedup 1.0000x reference)
import functools

import jax
import jax.numpy as jnp
from jax import lax
from jax.experimental import pallas as pl
from jax.experimental.pallas import tpu as pltpu
from jax.experimental.pallas import tpu_sc as plsc

_N = 10000
_E = 160000
_NSUB = 16
_NCORE = 2
_CHUNK = 128
_EPT = 80 * _CHUNK
_EPAD = _NSUB * _EPT
_NROWS = 10112
_DROWS = 640
_DQ = 80

_SC_PARAMS = pltpu.CompilerParams(needs_layout_passes=False,
                                  use_tc_tiling_on_sc=False)
_MESH = plsc.VectorSubcoreMesh(core_axis_name="c", subcore_axis_name="s",
                               num_cores=_NCORE, num_subcores=_NSUB)


def _mm_body(x_ref, w_ref, o_ref, *, act):
    acc = jnp.dot(x_ref[...], w_ref[...], preferred_element_type=jnp.float32)
    if act == "relu":
        acc = jnp.maximum(acc, 0.0)
    elif act == "leaky1":
        acc = jnp.maximum(acc, 0.01 * acc)
    elif act == "leaky2":
        acc = jnp.maximum(acc, 0.02 * acc)
    o_ref[...] = acc


def _round_up(x, m):
    return (x + m - 1) // m * m


def matmul_bias(x, w, b, act=None, block_m=1000):
    M, K = x.shape
    N = w.shape[1]
    Kp = _round_up(K + 1, 128)
    Np = _round_up(N, 128)
    Mp = _round_up(M, block_m)
    xp = jnp.zeros((Mp, Kp), jnp.float32)
    xp = xp.at[:M, :K].set(x)
    xp = xp.at[:, K].set(1.0)
    wp = jnp.zeros((Kp, Np), jnp.float32)
    wp = wp.at[:K, :N].set(w)
    wp = wp.at[K, :N].set(b)
    grid = (Mp // block_m,)
    out = pl.pallas_call(
        functools.partial(_mm_body, act=act),
        grid=grid,
        in_specs=[
            pl.BlockSpec((block_m, Kp), lambda i: (i, 0)),
            pl.BlockSpec((Kp, Np), lambda i: (0, 0)),
        ],
        out_specs=pl.BlockSpec((block_m, Np), lambda i: (i, 0)),
        out_shape=jax.ShapeDtypeStruct((Mp, Np), jnp.float32),
    )(xp, wp)
    return out[:M, :N]


def _agg_body(zq_hbm, src_hbm, dst_hbm, al_hbm, out_hbm,
              src_v, dst_v, al_v, albuf,
              sidx0, sidx1, didx, gbuf0, gbuf1, agg_sh, sem0, sem1, *, dq):
    c = lax.axis_index("c")
    s = lax.axis_index("s")
    base_e = s * _EPT
    zero16 = jnp.zeros((16,), jnp.float32)
    npairs = _EPT // _CHUNK // 2

    pltpu.sync_copy(src_hbm.at[pl.ds(base_e, _EPT)], src_v)
    pltpu.sync_copy(dst_hbm.at[pl.ds(base_e, _EPT)], dst_v)
    pltpu.sync_copy(al_hbm.at[pl.ds(base_e, _EPT)], al_v)

    def _zg(j, carry):
        for cb in range(dq // 16):
            gbuf0[j, pl.ds(cb * 16, 16)] = zero16
        return carry
    lax.fori_loop(0, _CHUNK, _zg, 0)

    rows_per_tile = _NROWS // _NSUB
    for j in range(4):
        pltpu.sync_copy(gbuf0, agg_sh.at[pl.ds(s * rows_per_tile + j * _CHUNK,
                                               _CHUNK)])
    pltpu.sync_copy(gbuf0.at[pl.ds(0, rows_per_tile - 4 * _CHUNK)],
                    agg_sh.at[pl.ds(s * rows_per_tile + 4 * _CHUNK,
                                    rows_per_tile - 4 * _CHUNK)])

    def _padj(i, carry):
        ds16 = pl.ds(i * 16, 16)
        src_v[ds16] = src_v[ds16] + c * _N
        return carry
    lax.fori_loop(0, _EPT // 16, _padj, 0)

    plsc.subcore_barrier()

    bufs = ((sidx0, gbuf0, sem0), (sidx1, gbuf1, sem1))

    def _stage(k, sidx_b):
        eb = k * _CHUNK
        for jj in range(_CHUNK // 16):
            sidx_b[pl.ds(jj * 16, 16)] = src_v[pl.ds(eb + jj * 16, 16)]

    for b in range(2):
        _stage(b, bufs[b][0])
        pltpu.async_copy(zq_hbm.at[bufs[b][0]], bufs[b][1], bufs[b][2])

    def _pair(kk, carry):
        for b in range(2):
            sidx_b, gbuf_b, sem_b = bufs[b]
            k = kk * 2 + b
            eb = k * _CHUNK
            pltpu.make_async_copy(zq_hbm.at[sidx_b], gbuf_b, sem_b).wait()
            for jj in range(_CHUNK // 16):
                ds16 = pl.ds(eb + jj * 16, 16)
                didx[pl.ds(jj * 16, 16)] = dst_v[ds16]
                albuf[pl.ds(jj * 16, 16)] = al_v[ds16]

            def _scale(j2, carry2, gbuf_b=gbuf_b):
                for u in range(2):
                    j = j2 * 2 + u
                    sp = plsc.load_gather(
                        albuf, [jnp.broadcast_to(j, (16,)).astype(jnp.int32)])
                    for cb in range(dq // 16):
                        dcb = pl.ds(cb * 16, 16)
                        gbuf_b[j, dcb] = gbuf_b[j, dcb] * sp
                return carry2
            lax.fori_loop(0, _CHUNK // 2, _scale, 0)

            pltpu.sync_copy(gbuf_b, agg_sh.at[didx], add=True)

            @pl.when(kk < npairs - 1)
            def _(k=k, sidx_b=sidx_b, gbuf_b=gbuf_b, sem_b=sem_b):
                _stage(k + 2, sidx_b)
                pltpu.async_copy(zq_hbm.at[sidx_b], gbuf_b, sem_b)
        return carry
    lax.fori_loop(0, npairs, _pair, 0)

    plsc.subcore_barrier()

    out_rows = _NROWS // _NSUB
    pltpu.sync_copy(agg_sh.at[pl.ds(s * out_rows, out_rows)],
                    out_hbm.at[c, pl.ds(s * out_rows, out_rows)])


def _agg_call(zq, srcp, dstp, alpha, dq):
    fn = pl.kernel(
        functools.partial(_agg_body, dq=dq),
        out_type=jax.ShapeDtypeStruct((_NCORE, _NROWS, dq), jnp.float32),
        mesh=_MESH,
        scratch_types=[
            pltpu.VMEM((_EPT,), jnp.int32),
            pltpu.VMEM((_EPT,), jnp.int32),
            pltpu.VMEM((_EPT,), jnp.float32),
            pltpu.VMEM((_CHUNK,), jnp.float32),
            pltpu.VMEM((_CHUNK,), jnp.int32),
            pltpu.VMEM((_CHUNK,), jnp.int32),
            pltpu.VMEM((_CHUNK,), jnp.int32),
            pltpu.VMEM((_CHUNK, dq), jnp.float32),
            pltpu.VMEM((_CHUNK, dq), jnp.float32),
            pltpu.VMEM_SHARED((_NROWS, dq), jnp.float32),
            pltpu.SemaphoreType.DMA,
            pltpu.SemaphoreType.DMA,
        ],
        compiler_params=_SC_PARAMS,
    )
    return fn(zq, srcp, dstp, alpha)


_EPT2 = _EPT // 2


def _den_body(s1_hbm, s2_hbm, src_hbm, dst_hbm, ewc_hbm, al_hbm, den_hbm,
              s1_v, s2_v, src_v, dst_v, ewc_v, al_v, den_v, ridx, den_sh):
    c = lax.axis_index("c")
    s = lax.axis_index("s")
    base_e = (c * _NSUB + s) * _EPT2
    zero16 = jnp.zeros((16,), jnp.float32)
    iota16 = lax.iota(jnp.int32, 16)

    pltpu.sync_copy(s1_hbm, s1_v.at[pl.ds(0, _N)])
    pltpu.sync_copy(s2_hbm, s2_v.at[pl.ds(0, _N)])
    s1_v[pl.ds(_N, 16)] = zero16
    s2_v[pl.ds(_N, 16)] = zero16
    pltpu.sync_copy(src_hbm.at[pl.ds(base_e, _EPT2)], src_v)
    pltpu.sync_copy(dst_hbm.at[pl.ds(base_e, _EPT2)], dst_v)
    pltpu.sync_copy(ewc_hbm.at[pl.ds(base_e, _EPT2)], ewc_v)

    def _zden(i, carry):
        den_v[i, :] = zero16
        return carry
    lax.fori_loop(0, _DROWS, _zden, 0)

    @pl.when(s == 0)
    def _():
        pltpu.sync_copy(den_v, den_sh)

    def _p1(i, carry):
        ds16 = pl.ds(i * 16, 16)
        srci = src_v[ds16]
        dsti = dst_v[ds16]
        e = (plsc.load_gather(s1_v, [srci]) + plsc.load_gather(s2_v, [dsti])
             + ewc_v[ds16])
        e = jnp.maximum(e, 0.01 * e)
        ex = jnp.exp(e)
        al_v[ds16] = ex
        plsc.addupdate_scatter(den_v, [dsti >> 4, dsti & 15], ex)
        return carry
    lax.fori_loop(0, _EPT2 // 16, _p1, 0)

    pltpu.sync_copy(al_v, al_hbm.at[pl.ds(base_e, _EPT2)])

    plsc.subcore_barrier()
    for j in range(_DROWS // _CHUNK):
        for jj in range(_CHUNK // 16):
            ridx[pl.ds(jj * 16, 16)] = iota16 + (j * _CHUNK + jj * 16)
        pltpu.sync_copy(den_v.at[pl.ds(j * _CHUNK, _CHUNK)],
                        den_sh.at[ridx], add=True)
    plsc.subcore_barrier()

    den_rows = _DROWS // _NSUB
    pltpu.sync_copy(den_sh.at[pl.ds(s * den_rows, den_rows)],
                    den_hbm.at[c, pl.ds(s * den_rows, den_rows)])


def _den_call(s1, s2, srcp, dstp, ewcp):
    fn = pl.kernel(
        _den_body,
        out_type=[jax.ShapeDtypeStruct((_EPAD,), jnp.float32),
                  jax.ShapeDtypeStruct((_NCORE, _DROWS, 16), jnp.float32)],
        mesh=_MESH,
        scratch_types=[
            pltpu.VMEM((_N + 16,), jnp.float32),
            pltpu.VMEM((_N + 16,), jnp.float32),
            pltpu.VMEM((_EPT2,), jnp.int32),
            pltpu.VMEM((_EPT2,), jnp.int32),
            pltpu.VMEM((_EPT2,), jnp.float32),
            pltpu.VMEM((_EPT2,), jnp.float32),
            pltpu.VMEM((_DROWS, 16), jnp.float32),
            pltpu.VMEM((_CHUNK,), jnp.int32),
            pltpu.VMEM_SHARED((_DROWS, 16), jnp.float32),
        ],
        compiler_params=_SC_PARAMS,
    )
    return fn(s1, s2, srcp, dstp, ewcp)


def _gat_layer(h, srcp, dstp, ewcp, snorm_n, p):
    n, d = h.shape
    ws1 = p['Wa'][:d, 0]
    ws2 = p['Wa'][d:2 * d, 0]
    w_big = jnp.concatenate([p['Wf'], (p['Wf'] @ ws1)[:, None],
                             (p['Wf'] @ ws2)[:, None], p['Ws']], axis=1)
    b_big = jnp.concatenate([p['bf'], (p['bf'] @ ws1 + p['ba'][0])[None],
                             (p['bf'] @ ws2)[None], p['bs']])
    big = matmul_bias(h, w_big, b_big)
    z = big[:, :d]
    s1 = big[:, d]
    s2 = big[:, d + 1]
    hs = big[:, d + 2:]

    q = (d + 3) // 4
    bounds = [0, q, 2 * q, 3 * q, d]

    def quarter(i):
        x = z[:, bounds[i]:bounds[i + 1]]
        return jnp.pad(x, ((0, 0), (0, _DQ - x.shape[1])))

    za = jnp.concatenate([quarter(0), quarter(2)], axis=0)
    zb = jnp.concatenate([quarter(1), quarter(3)], axis=0)
    alpha, den2 = _den_call(s1, s2, srcp, dstp, ewcp)
    outa = _agg_call(za, srcp, dstp, alpha, _DQ)
    outb = _agg_call(zb, srcp, dstp, alpha, _DQ)
    agg = jnp.concatenate([
        outa[0, :_N, :bounds[1] - bounds[0]],
        outb[0, :_N, :bounds[2] - bounds[1]],
        outa[1, :_N, :bounds[3] - bounds[2]],
        outb[1, :_N, :bounds[4] - bounds[3]],
    ], axis=1)
    den = (den2[0] + den2[1]).reshape(-1)[:_N, None]
    scale = snorm_n / (den + 1e-9)
    return jnp.maximum(agg * scale + hs, 0.0)


def kernel(feats, edge_index, e_w, snorm_n, gt, maps_emb, eps, params):
    src = edge_index[0].astype(jnp.int32)
    dst = edge_index[1].astype(jnp.int32)
    ew = e_w[:, 0]

    srcp = jnp.zeros((_EPAD,), jnp.int32).at[:_E].set(src)
    dstp = jnp.full((_EPAD,), _N, jnp.int32).at[:_E].set(dst)

    def _ewc(p, d, att_ew):
        if not att_ew:
            return jnp.zeros((_EPAD,), jnp.float32)
        cc = jnp.sum(p['Wa'][2 * d:, 0])
        return jnp.zeros((_EPAD,), jnp.float32).at[:_E].set(ew * cc)

    h_emb = matmul_bias(feats, params['emb']['W'], params['emb']['b'])
    x = jnp.concatenate([maps_emb, h_emb, gt], axis=-1)
    de = x.shape[1]
    h = _gat_layer(x, srcp, dstp, _ewc(params['enc1'], de, True), snorm_n,
                   params['enc1'])
    h = _gat_layer(h, srcp, dstp, _ewc(params['enc2'], de, True), snorm_n,
                   params['enc2'])

    he = jnp.concatenate([h, gt], axis=-1)
    pe = params['mlp_enc']
    hl = matmul_bias(he, pe['Wl'], pe['bl'], act="leaky1")
    w_mu_lv = jnp.concatenate([pe['Wmu'], pe['Wlv']], axis=1)
    b_mu_lv = jnp.concatenate([pe['bmu'], pe['blv']])
    mu_lv = matmul_bias(hl, w_mu_lv, b_mu_lv)
    zdim = pe['Wmu'].shape[1]
    mu = mu_lv[:, :zdim]
    log_var = mu_lv[:, zdim:]
    zlat = mu + eps * jnp.exp(0.5 * log_var)

    xd = jnp.concatenate([maps_emb, h_emb, zlat], axis=-1)
    zero_ewc = jnp.zeros((_EPAD,), jnp.float32)
    hd = _gat_layer(xd, srcp, dstp, zero_ewc, snorm_n, params['dec1'])
    hd = _gat_layer(hd, srcp, dstp, zero_ewc, snorm_n, params['dec2'])

    hdz = jnp.concatenate([hd, zlat], axis=-1)
    pd_ = params['mlp_dec']
    h0 = matmul_bias(hdz, pd_['W0'], pd_['b0'], act="leaky2")
    recon = matmul_bias(h0, pd_['W1'], pd_['b1'])
    return recon, mu, log_var


# speedup vs baseline: 1.0189x; 1.0189x over previous
import functools

import jax
import jax.numpy as jnp
from jax import lax
from jax.experimental import pallas as pl
from jax.experimental.pallas import tpu as pltpu
from jax.experimental.pallas import tpu_sc as plsc

_N = 10000
_E = 160000
_NSUB = 16
_NCORE = 2
_CHUNK = 128
_EPT = 80 * _CHUNK
_EPAD = _NSUB * _EPT
_NROWS = 10112
_DROWS = 640
_DQ = 80

_SC_PARAMS = pltpu.CompilerParams(needs_layout_passes=False,
                                  use_tc_tiling_on_sc=False)
_MESH = plsc.VectorSubcoreMesh(core_axis_name="c", subcore_axis_name="s",
                               num_cores=_NCORE, num_subcores=_NSUB)


def _mm_body(x_ref, w_ref, o_ref, *, act):
    acc = jnp.dot(x_ref[...], w_ref[...], preferred_element_type=jnp.float32)
    if act == "relu":
        acc = jnp.maximum(acc, 0.0)
    elif act == "leaky1":
        acc = jnp.maximum(acc, 0.01 * acc)
    elif act == "leaky2":
        acc = jnp.maximum(acc, 0.02 * acc)
    o_ref[...] = acc


def _round_up(x, m):
    return (x + m - 1) // m * m


def matmul_bias(x, w, b, act=None, block_m=1000):
    M, K = x.shape
    N = w.shape[1]
    Kp = _round_up(K + 1, 128)
    Np = _round_up(N, 128)
    Mp = _round_up(M, block_m)
    xp = jnp.zeros((Mp, Kp), jnp.float32)
    xp = xp.at[:M, :K].set(x)
    xp = xp.at[:, K].set(1.0)
    wp = jnp.zeros((Kp, Np), jnp.float32)
    wp = wp.at[:K, :N].set(w)
    wp = wp.at[K, :N].set(b)
    grid = (Mp // block_m,)
    out = pl.pallas_call(
        functools.partial(_mm_body, act=act),
        grid=grid,
        in_specs=[
            pl.BlockSpec((block_m, Kp), lambda i: (i, 0)),
            pl.BlockSpec((Kp, Np), lambda i: (0, 0)),
        ],
        out_specs=pl.BlockSpec((block_m, Np), lambda i: (i, 0)),
        out_shape=jax.ShapeDtypeStruct((Mp, Np), jnp.float32),
    )(xp, wp)
    return out[:M, :N]


def _alpha_body(s1_hbm, s2_hbm, src_hbm, dst_hbm, ewc_hbm, al_hbm,
                s1_v, s2_v, src_v, dst_v, ewc_v, al_v, den_v, ridx,
                den_sh):
    s = lax.axis_index("s")
    base_e = s * _EPT
    zero16 = jnp.zeros((16,), jnp.float32)
    iota16 = lax.iota(jnp.int32, 16)

    pltpu.sync_copy(s1_hbm, s1_v)
    pltpu.sync_copy(s2_hbm, s2_v)
    pltpu.sync_copy(src_hbm.at[pl.ds(base_e, _EPT)], src_v)
    pltpu.sync_copy(dst_hbm.at[pl.ds(base_e, _EPT)], dst_v)
    pltpu.sync_copy(ewc_hbm.at[pl.ds(base_e, _EPT)], ewc_v)

    def _zden(i, carry):
        den_v[i, :] = zero16
        return carry
    lax.fori_loop(0, _DROWS, _zden, 0)

    @pl.when(s == 0)
    def _():
        pltpu.sync_copy(den_v, den_sh)

    def _p1(i, carry):
        ds16 = pl.ds(i * 16, 16)
        srci = src_v[ds16]
        dsti = dst_v[ds16]
        e = (plsc.load_gather(s1_v, [srci]) + plsc.load_gather(s2_v, [dsti])
             + ewc_v[ds16])
        e = jnp.maximum(e, 0.01 * e)
        ex = jnp.exp(e)
        al_v[ds16] = ex
        plsc.addupdate_scatter(den_v, [dsti >> 4, dsti & 15], ex)
        return carry
    lax.fori_loop(0, _EPT // 16, _p1, 0)

    plsc.subcore_barrier()
    for j in range(_DROWS // _CHUNK):
        for jj in range(_CHUNK // 16):
            ridx[pl.ds(jj * 16, 16)] = iota16 + (j * _CHUNK + jj * 16)
        pltpu.sync_copy(den_v.at[pl.ds(j * _CHUNK, _CHUNK)],
                        den_sh.at[ridx], add=True)
    plsc.subcore_barrier()
    pltpu.sync_copy(den_sh, den_v)

    def _p2(i, carry):
        ds16 = pl.ds(i * 16, 16)
        dsti = dst_v[ds16]
        den16 = plsc.load_gather(den_v, [dsti >> 4, dsti & 15])
        al_v[ds16] = al_v[ds16] / (den16 + 1e-9)
        return carry
    lax.fori_loop(0, _EPT // 16, _p2, 0)

    pltpu.sync_copy(al_v, al_hbm.at[pl.ds(base_e, _EPT)])


def _alpha_call(s1, s2, srcp, dstp, ewcp):
    fn = pl.kernel(
        _alpha_body,
        out_type=jax.ShapeDtypeStruct((_EPAD,), jnp.float32),
        mesh=_MESH,
        scratch_types=[
            pltpu.VMEM((_N,), jnp.float32),
            pltpu.VMEM((_N,), jnp.float32),
            pltpu.VMEM((_EPT,), jnp.int32),
            pltpu.VMEM((_EPT,), jnp.int32),
            pltpu.VMEM((_EPT,), jnp.float32),
            pltpu.VMEM((_EPT,), jnp.float32),
            pltpu.VMEM((_DROWS, 16), jnp.float32),
            pltpu.VMEM((_CHUNK,), jnp.int32),
            pltpu.VMEM_SHARED((_DROWS, 16), jnp.float32),
        ],
        compiler_params=_SC_PARAMS,
    )
    return fn(s1, s2, srcp, dstp, ewcp)


def _agg_body(zq_hbm, src_hbm, dst_hbm, al_hbm, out_hbm,
              src_v, dst_v, al_v, albuf, sidx0, sidx1, didx,
              gbuf0, gbuf1, agg_sh, sem0, sem1):
    c = lax.axis_index("c")
    s = lax.axis_index("s")
    base_e = s * _EPT
    zero16 = jnp.zeros((16,), jnp.float32)
    npairs = _EPT // _CHUNK // 2

    pltpu.sync_copy(src_hbm.at[pl.ds(base_e, _EPT)], src_v)
    pltpu.sync_copy(dst_hbm.at[pl.ds(base_e, _EPT)], dst_v)
    pltpu.sync_copy(al_hbm.at[pl.ds(base_e, _EPT)], al_v)

    def _zg(j, carry):
        for cb in range(_DQ // 16):
            gbuf0[j, pl.ds(cb * 16, 16)] = zero16
        return carry
    lax.fori_loop(0, _CHUNK, _zg, 0)

    rows_per_tile = _NROWS // _NSUB
    for j in range(4):
        pltpu.sync_copy(gbuf0, agg_sh.at[pl.ds(s * rows_per_tile + j * _CHUNK,
                                               _CHUNK)])
    pltpu.sync_copy(gbuf0.at[pl.ds(0, rows_per_tile - 4 * _CHUNK)],
                    agg_sh.at[pl.ds(s * rows_per_tile + 4 * _CHUNK,
                                    rows_per_tile - 4 * _CHUNK)])

    def _padj(i, carry):
        ds16 = pl.ds(i * 16, 16)
        src_v[ds16] = src_v[ds16] + c * _N
        return carry
    lax.fori_loop(0, _EPT // 16, _padj, 0)

    plsc.subcore_barrier()

    bufs = ((sidx0, gbuf0, sem0), (sidx1, gbuf1, sem1))

    def _stage(k, sidx_b):
        eb = k * _CHUNK
        for jj in range(_CHUNK // 16):
            sidx_b[pl.ds(jj * 16, 16)] = src_v[pl.ds(eb + jj * 16, 16)]

    for b in range(2):
        _stage(b, bufs[b][0])
        pltpu.async_copy(zq_hbm.at[bufs[b][0]], bufs[b][1], bufs[b][2])

    def _pair(kk, carry):
        for b in range(2):
            sidx_b, gbuf_b, sem_b = bufs[b]
            k = kk * 2 + b
            eb = k * _CHUNK
            pltpu.make_async_copy(zq_hbm.at[sidx_b], gbuf_b, sem_b).wait()
            for jj in range(_CHUNK // 16):
                ds16 = pl.ds(eb + jj * 16, 16)
                didx[pl.ds(jj * 16, 16)] = dst_v[ds16]
                albuf[pl.ds(jj * 16, 16)] = al_v[ds16]

            def _scale(j2, carry2, gbuf_b=gbuf_b):
                for u in range(2):
                    j = j2 * 2 + u
                    sp = plsc.load_gather(
                        albuf, [jnp.broadcast_to(j, (16,)).astype(jnp.int32)])
                    for cb in range(_DQ // 16):
                        dcb = pl.ds(cb * 16, 16)
                        gbuf_b[j, dcb] = gbuf_b[j, dcb] * sp
                return carry2
            lax.fori_loop(0, _CHUNK // 2, _scale, 0)

            pltpu.sync_copy(gbuf_b, agg_sh.at[didx], add=True)

            @pl.when(kk < npairs - 1)
            def _(k=k, sidx_b=sidx_b, gbuf_b=gbuf_b, sem_b=sem_b):
                _stage(k + 2, sidx_b)
                pltpu.async_copy(zq_hbm.at[sidx_b], gbuf_b, sem_b)
        return carry
    lax.fori_loop(0, npairs, _pair, 0)

    plsc.subcore_barrier()

    out_rows = _NROWS // _NSUB
    pltpu.sync_copy(agg_sh.at[pl.ds(s * out_rows, out_rows)],
                    out_hbm.at[c, pl.ds(s * out_rows, out_rows)])


def _agg_call(zq, srcp, dstp, alpha):
    fn = pl.kernel(
        _agg_body,
        out_type=jax.ShapeDtypeStruct((_NCORE, _NROWS, _DQ), jnp.float32),
        mesh=_MESH,
        scratch_types=[
            pltpu.VMEM((_EPT,), jnp.int32),
            pltpu.VMEM((_EPT,), jnp.int32),
            pltpu.VMEM((_EPT,), jnp.float32),
            pltpu.VMEM((_CHUNK,), jnp.float32),
            pltpu.VMEM((_CHUNK,), jnp.int32),
            pltpu.VMEM((_CHUNK,), jnp.int32),
            pltpu.VMEM((_CHUNK,), jnp.int32),
            pltpu.VMEM((_CHUNK, _DQ), jnp.float32),
            pltpu.VMEM((_CHUNK, _DQ), jnp.float32),
            pltpu.VMEM_SHARED((_NROWS, _DQ), jnp.float32),
            pltpu.SemaphoreType.DMA,
            pltpu.SemaphoreType.DMA,
        ],
        compiler_params=_SC_PARAMS,
    )
    return fn(zq, srcp, dstp, alpha)


def _gat_layer(h, srcp, dstp, ewcp, snorm_n, p):
    n, d = h.shape
    ws1 = p['Wa'][:d, 0]
    ws2 = p['Wa'][d:2 * d, 0]
    w_big = jnp.concatenate([p['Wf'], (p['Wf'] @ ws1)[:, None],
                             (p['Wf'] @ ws2)[:, None], p['Ws']], axis=1)
    b_big = jnp.concatenate([p['bf'], (p['bf'] @ ws1 + p['ba'][0])[None],
                             (p['bf'] @ ws2)[None], p['bs']])
    big = matmul_bias(h, w_big, b_big)
    z = big[:, :d]
    s1 = big[:, d]
    s2 = big[:, d + 1]
    hs = big[:, d + 2:]

    alpha = _alpha_call(s1, s2, srcp, dstp, ewcp)

    q = (d + 3) // 4
    bounds = [0, q, 2 * q, 3 * q, d]

    def quarter(i):
        x = z[:, bounds[i]:bounds[i + 1]]
        return jnp.pad(x, ((0, 0), (0, _DQ - x.shape[1])))

    za = jnp.concatenate([quarter(0), quarter(2)], axis=0)
    zb = jnp.concatenate([quarter(1), quarter(3)], axis=0)
    outa = _agg_call(za, srcp, dstp, alpha)
    outb = _agg_call(zb, srcp, dstp, alpha)
    agg = jnp.concatenate([
        outa[0, :_N, :bounds[1] - bounds[0]],
        outb[0, :_N, :bounds[2] - bounds[1]],
        outa[1, :_N, :bounds[3] - bounds[2]],
        outb[1, :_N, :bounds[4] - bounds[3]],
    ], axis=1)
    return jnp.maximum(agg * snorm_n + hs, 0.0)


def kernel(feats, edge_index, e_w, snorm_n, gt, maps_emb, eps, params):
    src = edge_index[0].astype(jnp.int32)
    dst = edge_index[1].astype(jnp.int32)
    ew = e_w[:, 0]

    srcp = jnp.zeros((_EPAD,), jnp.int32).at[:_E].set(src)
    dstp = jnp.full((_EPAD,), _N, jnp.int32).at[:_E].set(dst)

    def _ewc(p, d, att_ew):
        if not att_ew:
            return jnp.zeros((_EPAD,), jnp.float32)
        cc = jnp.sum(p['Wa'][2 * d:, 0])
        return jnp.zeros((_EPAD,), jnp.float32).at[:_E].set(ew * cc)

    h_emb = matmul_bias(feats, params['emb']['W'], params['emb']['b'])
    x = jnp.concatenate([maps_emb, h_emb, gt], axis=-1)
    de = x.shape[1]
    h = _gat_layer(x, srcp, dstp, _ewc(params['enc1'], de, True), snorm_n,
                   params['enc1'])
    h = _gat_layer(h, srcp, dstp, _ewc(params['enc2'], de, True), snorm_n,
                   params['enc2'])

    he = jnp.concatenate([h, gt], axis=-1)
    pe = params['mlp_enc']
    hl = matmul_bias(he, pe['Wl'], pe['bl'], act="leaky1")
    w_mu_lv = jnp.concatenate([pe['Wmu'], pe['Wlv']], axis=1)
    b_mu_lv = jnp.concatenate([pe['bmu'], pe['blv']])
    mu_lv = matmul_bias(hl, w_mu_lv, b_mu_lv)
    zdim = pe['Wmu'].shape[1]
    mu = mu_lv[:, :zdim]
    log_var = mu_lv[:, zdim:]
    zlat = mu + eps * jnp.exp(0.5 * log_var)

    xd = jnp.concatenate([maps_emb, h_emb, zlat], axis=-1)
    zero_ewc = jnp.zeros((_EPAD,), jnp.float32)
    hd = _gat_layer(xd, srcp, dstp, zero_ewc, snorm_n, params['dec1'])
    hd = _gat_layer(hd, srcp, dstp, zero_ewc, snorm_n, params['dec2'])

    hdz = jnp.concatenate([hd, zlat], axis=-1)
    pd_ = params['mlp_dec']
    h0 = matmul_bias(hdz, pd_['W0'], pd_['b0'], act="leaky2")
    recon = matmul_bias(h0, pd_['W1'], pd_['b1'])
    return recon, mu, log_var
